# baseline (speedup 1.0000x reference)
import jax
import jax.numpy as jnp
from jax.experimental import pallas as pl
from jax.experimental.pallas import tpu as pltpu

N, W = 131072, 64
BLK = 4096
NB = N // BLK
OUT_ROWS = N // 128


def _body(key_ref, s_ref, mem_ref, out_ref, acc_ref):
    i = pl.program_id(0)
    mb = mem_ref[...]
    kv = key_ref[...]
    dot = jnp.sum(mb * kv, axis=1)
    sq = jnp.sum(mb * mb, axis=1)
    n1 = jnp.sqrt(jnp.sum(kv * kv))
    denom = jnp.maximum(n1 * jnp.sqrt(sq), 1e-8)
    e = jnp.exp(dot / denom * s_ref[0])
    bsum = jnp.sum(e)

    @pl.when(i == 0)
    def _():
        acc_ref[0] = bsum

    @pl.when(i > 0)
    def _():
        acc_ref[0] = acc_ref[0] + bsum

    out_ref[pl.ds(i * (BLK // 128), BLK // 128), :] = e.reshape(BLK // 128, 128)

    @pl.when(i == NB - 1)
    def _():
        out_ref[...] = out_ref[...] * (1.0 / acc_ref[0])


def kernel(write_key, write_strength, memory):
    out = pl.pallas_call(
        _body,
        grid=(NB,),
        in_specs=[
            pl.BlockSpec((1, W), lambda i: (0, 0)),
            pl.BlockSpec(memory_space=pltpu.SMEM),
            pl.BlockSpec((BLK, W), lambda i: (i, 0)),
        ],
        out_specs=pl.BlockSpec((OUT_ROWS, 128), lambda i: (0, 0)),
        out_shape=jax.ShapeDtypeStruct((OUT_ROWS, 128), jnp.float32),
        scratch_shapes=[pltpu.SMEM((1,), jnp.float32)],
    )(write_key, write_strength, memory)
    return out.reshape(N)
